# baseline (speedup 1.0000x reference)
import functools

import jax
import jax.numpy as jnp
from jax import lax
from jax.experimental import pallas as pl
from jax.experimental.pallas import tpu as pltpu
from jax.experimental.pallas import tpu_sc as plsc

N_NODES = 10000
N_EDGES = 320000
D = 128
NUM_RELS = 16
NUM_BASES = 8

NW = 32
CH = 80
NCH = 128
EPW = NCH * CH
E_PAD = NW * EPW
NPAIR = NCH // 2
N_PAD = 10240
RPT = N_PAD // 16
DEG_BATCH = 8
DST_BITS = 14


def _proj_body(wc_ref, w_ref, h_ref, out_ref):
    r = pl.program_id(0)
    rel_w = wc_ref[r, 0] * w_ref[0]
    for b in range(1, NUM_BASES):
        rel_w = rel_w + wc_ref[r, b] * w_ref[b]
    out_ref[0] = jnp.dot(h_ref[...], rel_w, preferred_element_type=jnp.float32)


def _proj(h, weight, w_comp):
    return pl.pallas_call(
        _proj_body,
        grid=(NUM_RELS,),
        in_specs=[
            pl.BlockSpec(memory_space=pltpu.SMEM),
            pl.BlockSpec((NUM_BASES, D, D), lambda r: (0, 0, 0)),
            pl.BlockSpec((N_NODES, D), lambda r: (0, 0)),
        ],
        out_specs=pl.BlockSpec((1, N_NODES, D), lambda r: (r, 0, 0)),
        out_shape=jax.ShapeDtypeStruct((NUM_RELS, N_NODES, D), jnp.float32),
    )(w_comp, weight, h)


def _sc_agg_body(proj_hbm, pidx_hbm, zrow_hbm,
                 part_out, deg_out,
                 pidx_v, gc_a, gc_b, dc_a, dc_b, dst8_v,
                 rows_a, rows_b, acc_sh,
                 sem_ga, sem_gb, sem_sa, sem_sb, sem_deg):
    cid = lax.axis_index("c")
    sid = lax.axis_index("s")
    wid = sid * 2 + cid

    pltpu.sync_copy(zrow_hbm, acc_sh.at[pl.ds(sid * RPT, RPT)])
    pltpu.sync_copy(pidx_hbm.at[wid], pidx_v)
    plsc.subcore_barrier()

    def unpack(i, gc, dc):
        for j in range(CH // 16):
            pv = pidx_v[i, pl.ds(j * 16, 16)]
            gc[pl.ds(j * 16, 16)] = lax.shift_right_logical(pv, DST_BITS)
            dc[pl.ds(j * 16, 16)] = lax.bitwise_and(pv, (1 << DST_BITS) - 1)

    def wait_dma(src, buf, sem):
        pltpu.make_async_copy(src.at[pl.ds(0, CH)], buf, sem).wait()

    def wait_gather(buf, sem):
        pltpu.make_async_copy(proj_hbm.at[pl.ds(0, CH)], buf, sem).wait()

    def wait_scatter(buf, sem):
        pltpu.make_async_copy(buf, acc_sh.at[pl.ds(0, CH)], sem).wait()

    def body(i, carry):
        unpack(i, gc_a, dc_a)
        pltpu.async_copy(proj_hbm.at[gc_a], rows_a, sem_ga).wait()
        pltpu.sync_copy(rows_a, acc_sh.at[dc_a], add=True)
        return carry

    lax.fori_loop(0, NCH, body, 0)
    plsc.subcore_barrier()

    pltpu.sync_copy(acc_sh.at[pl.ds(sid * RPT, RPT)],
                    part_out.at[cid, pl.ds(sid * RPT, RPT)])
    plsc.subcore_barrier()

    pltpu.sync_copy(zrow_hbm, acc_sh.at[pl.ds(sid * RPT, RPT)])

    def fill_ones(i, carry):
        for j in range(D // 16):
            rows_a[i, pl.ds(j * 16, 16)] = jnp.ones((16,), jnp.float32)
        return carry

    lax.fori_loop(0, CH, fill_ones, 0)
    plsc.subcore_barrier()

    def deg_batch(g, carry):
        for k in range(DEG_BATCH):
            i = g * DEG_BATCH + k
            for j in range(CH // 16):
                pv = pidx_v[i, pl.ds(j * 16, 16)]
                dst8_v[k, pl.ds(j * 16, 16)] = lax.bitwise_and(
                    pv, (1 << DST_BITS) - 1)
        for k in range(DEG_BATCH):
            pltpu.async_copy(rows_a, acc_sh.at[dst8_v.at[k]],
                             sem_deg, add=True)
        for k in range(DEG_BATCH):
            wait_scatter(rows_a, sem_deg)
        return carry

    lax.fori_loop(0, NCH // DEG_BATCH, deg_batch, 0)
    plsc.subcore_barrier()

    pltpu.sync_copy(acc_sh.at[pl.ds(sid * RPT, RPT)],
                    deg_out.at[cid, pl.ds(sid * RPT, RPT)])


_sc_agg = functools.partial(
    pl.kernel,
    mesh=plsc.VectorSubcoreMesh(core_axis_name="c", subcore_axis_name="s"),
    out_type=[
        jax.ShapeDtypeStruct((2, N_PAD, D), jnp.float32),
        jax.ShapeDtypeStruct((2, N_PAD, D), jnp.float32),
    ],
    scratch_types=[
        pltpu.VMEM((NCH, CH), jnp.int32),
        pltpu.VMEM((CH,), jnp.int32),
        pltpu.VMEM((CH,), jnp.int32),
        pltpu.VMEM((CH,), jnp.int32),
        pltpu.VMEM((CH,), jnp.int32),
        pltpu.VMEM((DEG_BATCH, CH), jnp.int32),
        pltpu.VMEM((CH, D), jnp.float32),
        pltpu.VMEM((CH, D), jnp.float32),
        pltpu.VMEM_SHARED((N_PAD, D), jnp.float32),
        pltpu.SemaphoreType.DMA,
        pltpu.SemaphoreType.DMA,
        pltpu.SemaphoreType.DMA,
        pltpu.SemaphoreType.DMA,
        pltpu.SemaphoreType.DMA,
    ],
)(_sc_agg_body)


def _combine_body(part_ref, degp_ref, h_ref, slw_ref, out_ref):
    deg = degp_ref[0, :N_NODES] + degp_ref[1, :N_NODES]
    s = jnp.log(deg + 1.0)
    mean = jnp.sum(s) * (1.0 / (N_NODES * D))
    scale = s * (1.0 / mean)
    nei = part_ref[0, :N_NODES] + part_ref[1, :N_NODES]
    out_ref[...] = (
        jnp.dot(h_ref[...], slw_ref[...], preferred_element_type=jnp.float32)
        + nei * scale
    )


def _combine(part, degp, h, slw):
    return pl.pallas_call(
        _combine_body,
        out_shape=jax.ShapeDtypeStruct((N_NODES, D), jnp.float32),
    )(part, degp, h, slw)


def kernel(h, edge_index, edge_type, weight, w_comp, self_loop_weight):
    src = edge_index[0].astype(jnp.uint32)
    dst = edge_index[1].astype(jnp.uint32)
    gidx = edge_type.astype(jnp.uint32) * N_NODES + src

    pad = E_PAD - N_EDGES
    packed = (gidx << DST_BITS) | dst
    packed = jnp.concatenate(
        [packed, jnp.full((pad,), N_PAD - 1, jnp.uint32)])
    packed = lax.bitcast_convert_type(packed, jnp.int32).reshape(NW, NCH, CH)

    all_proj = _proj(h, weight, w_comp).reshape(NUM_RELS * N_NODES, D)

    zrow = jnp.zeros((RPT, D), jnp.float32)
    part, degp = _sc_agg(all_proj, packed, zrow)

    return _combine(part, degp, h, self_loop_weight)


# speedup vs baseline: 1.6953x; 1.6953x over previous
import functools

import jax
import jax.numpy as jnp
from jax import lax
from jax.experimental import pallas as pl
from jax.experimental.pallas import tpu as pltpu
from jax.experimental.pallas import tpu_sc as plsc

N_NODES = 10000
N_EDGES = 320000
D = 128
NUM_RELS = 16
NUM_BASES = 8

NW = 32
CHUNK = 80
NCHUNK = 126
EPW = NCHUNK * CHUNK
E_PAD = NW * EPW
NPAIR = NCHUNK // 2
N_PAD = 10240
RPT = N_PAD // 16


def _proj_body(wc_ref, w_ref, h_ref, out_ref):
    r = pl.program_id(0)
    rel_w = wc_ref[r, 0] * w_ref[0]
    for b in range(1, NUM_BASES):
        rel_w = rel_w + wc_ref[r, b] * w_ref[b]
    out_ref[0] = jnp.dot(h_ref[...], rel_w, preferred_element_type=jnp.float32)


def _proj(h, weight, w_comp):
    return pl.pallas_call(
        _proj_body,
        grid=(NUM_RELS,),
        in_specs=[
            pl.BlockSpec(memory_space=pltpu.SMEM),
            pl.BlockSpec((NUM_BASES, D, D), lambda r: (0, 0, 0)),
            pl.BlockSpec((N_NODES, D), lambda r: (0, 0)),
        ],
        out_specs=pl.BlockSpec((1, N_NODES, D), lambda r: (r, 0, 0)),
        out_shape=jax.ShapeDtypeStruct((NUM_RELS, N_NODES, D), jnp.float32),
    )(w_comp, weight, h)


def _sc_agg_body(proj_hbm, gidx_hbm, dst_hbm, zrow_hbm,
                 part_out, deg_out,
                 gi_a, gi_b, ds_a, ds_b, rows_a, rows_b, ones_v, acc_sh,
                 sem_ga, sem_gb):
    cid = lax.axis_index("c")
    sid = lax.axis_index("s")
    wid = sid * 2 + cid

    pltpu.sync_copy(zrow_hbm, acc_sh.at[pl.ds(sid * RPT, RPT)])

    def fill_ones(i, carry):
        for j in range(D // 16):
            ones_v[i, pl.ds(j * 16, 16)] = jnp.ones((16,), jnp.float32)
        return carry

    lax.fori_loop(0, CHUNK, fill_ones, 0)
    plsc.subcore_barrier()

    base0 = wid * EPW

    def load_idx(i, gi, ds):
        pltpu.sync_copy(gidx_hbm.at[pl.ds(base0 + i * CHUNK, CHUNK)], gi)
        pltpu.sync_copy(dst_hbm.at[pl.ds(base0 + i * CHUNK, CHUNK)], ds)

    def wait_gather(buf, sem):
        pltpu.make_async_copy(proj_hbm.at[pl.ds(0, CHUNK)], buf, sem).wait()

    load_idx(0, gi_a, ds_a)
    pltpu.async_copy(proj_hbm.at[gi_a], rows_a, sem_ga)

    def pair(p, carry):
        i0 = 2 * p
        load_idx(i0 + 1, gi_b, ds_b)
        wait_gather(rows_a, sem_ga)
        pltpu.async_copy(proj_hbm.at[gi_b], rows_b, sem_gb)
        pltpu.sync_copy(rows_a, acc_sh.at[ds_a], add=True)

        @pl.when(p < NPAIR - 1)
        def _():
            load_idx(i0 + 2, gi_a, ds_a)
            pltpu.async_copy(proj_hbm.at[gi_a], rows_a, sem_ga)

        wait_gather(rows_b, sem_gb)
        pltpu.sync_copy(rows_b, acc_sh.at[ds_b], add=True)
        return carry

    lax.fori_loop(0, NPAIR, pair, 0)
    plsc.subcore_barrier()

    pltpu.sync_copy(acc_sh.at[pl.ds(sid * RPT, RPT)],
                    part_out.at[cid, pl.ds(sid * RPT, RPT)])
    plsc.subcore_barrier()

    pltpu.sync_copy(zrow_hbm, acc_sh.at[pl.ds(sid * RPT, RPT)])
    plsc.subcore_barrier()

    def body2(i, carry):
        pltpu.sync_copy(dst_hbm.at[pl.ds(base0 + i * CHUNK, CHUNK)], ds_a)
        pltpu.sync_copy(ones_v, acc_sh.at[ds_a], add=True)
        return carry

    lax.fori_loop(0, NCHUNK, body2, 0)
    plsc.subcore_barrier()

    pltpu.sync_copy(acc_sh.at[pl.ds(sid * RPT, RPT)],
                    deg_out.at[cid, pl.ds(sid * RPT, RPT)])


_sc_agg = functools.partial(
    pl.kernel,
    mesh=plsc.VectorSubcoreMesh(core_axis_name="c", subcore_axis_name="s"),
    out_type=[
        jax.ShapeDtypeStruct((2, N_PAD, D), jnp.float32),
        jax.ShapeDtypeStruct((2, N_PAD, D), jnp.float32),
    ],
    scratch_types=[
        pltpu.VMEM((CHUNK,), jnp.int32),
        pltpu.VMEM((CHUNK,), jnp.int32),
        pltpu.VMEM((CHUNK,), jnp.int32),
        pltpu.VMEM((CHUNK,), jnp.int32),
        pltpu.VMEM((CHUNK, D), jnp.float32),
        pltpu.VMEM((CHUNK, D), jnp.float32),
        pltpu.VMEM((CHUNK, D), jnp.float32),
        pltpu.VMEM_SHARED((N_PAD, D), jnp.float32),
        pltpu.SemaphoreType.DMA,
        pltpu.SemaphoreType.DMA,
    ],
)(_sc_agg_body)


def _combine_body(part_ref, degp_ref, h_ref, slw_ref, out_ref):
    deg = degp_ref[0, :N_NODES] + degp_ref[1, :N_NODES]
    s = jnp.log(deg + 1.0)
    mean = jnp.sum(s) * (1.0 / (N_NODES * D))
    scale = s * (1.0 / mean)
    nei = part_ref[0, :N_NODES] + part_ref[1, :N_NODES]
    out_ref[...] = (
        jnp.dot(h_ref[...], slw_ref[...], preferred_element_type=jnp.float32)
        + nei * scale
    )


def _combine(part, degp, h, slw):
    return pl.pallas_call(
        _combine_body,
        out_shape=jax.ShapeDtypeStruct((N_NODES, D), jnp.float32),
    )(part, degp, h, slw)


def kernel(h, edge_index, edge_type, weight, w_comp, self_loop_weight):
    src = edge_index[0].astype(jnp.int32)
    dst = edge_index[1].astype(jnp.int32)
    gidx = edge_type.astype(jnp.int32) * N_NODES + src

    pad = E_PAD - N_EDGES
    gidx_p = jnp.concatenate([gidx, jnp.zeros((pad,), jnp.int32)])
    dst_p = jnp.concatenate([dst, jnp.full((pad,), N_PAD - 1, jnp.int32)])

    all_proj = _proj(h, weight, w_comp).reshape(NUM_RELS * N_NODES, D)

    zrow = jnp.zeros((RPT, D), jnp.float32)
    part, degp = _sc_agg(all_proj, gidx_p, dst_p, zrow)

    return _combine(part, degp, h, self_loop_weight)
